# Initial kernel scaffold; baseline (speedup 1.0000x reference)
import jax
import jax.numpy as jnp
from jax.experimental import pallas as pl


def kernel(src_node_features, neighbor_node_features, weight, agg_weight):
    raise NotImplementedError("write your pallas kernel here")


# speedup vs baseline: 1.3184x; 1.3184x over previous
import jax
import jax.numpy as jnp
from jax.experimental import pallas as pl
from jax.experimental.pallas import tpu as pltpu

_N = 10000
_K = 32
_D = 128
_BLOCK = 400


def _fused_body(src_ref, nbr_ref, w_ref, wa_ref, out_ref):
    agg = jnp.sum(nbr_ref[...], axis=1) * (1.0 / _K)
    self_h = jnp.dot(src_ref[...], w_ref[...], preferred_element_type=jnp.float32)
    nbr_h = jnp.dot(agg, wa_ref[...], preferred_element_type=jnp.float32)
    out_ref[...] = jnp.maximum(self_h + nbr_h, 0.0)


def kernel(src_node_features, neighbor_node_features, weight, agg_weight):
    n, d = src_node_features.shape
    k = neighbor_node_features.shape[1]
    grid = (n // _BLOCK,)
    return pl.pallas_call(
        _fused_body,
        grid=grid,
        in_specs=[
            pl.BlockSpec((_BLOCK, d), lambda i: (i, 0)),
            pl.BlockSpec((_BLOCK, k, d), lambda i: (i, 0, 0)),
            pl.BlockSpec((d, d), lambda i: (0, 0)),
            pl.BlockSpec((d, d), lambda i: (0, 0)),
        ],
        out_specs=pl.BlockSpec((_BLOCK, d), lambda i: (i, 0)),
        out_shape=jax.ShapeDtypeStruct((n, d), jnp.float32),
    )(src_node_features, neighbor_node_features, weight, agg_weight)
